# Initial kernel scaffold; baseline (speedup 1.0000x reference)
import functools

import jax
import jax.numpy as jnp
from jax import lax
from jax.experimental import pallas as pl
from jax.experimental.pallas import tpu as pltpu
from jax.experimental.pallas import tpu_sc as plsc

_NC = 2
_NS = 16
_NW = _NC * _NS
_LANES = 16
_CH = 512


@functools.lru_cache(maxsize=None)
def _sc_embed(n, bsz, vocab, d, ch):
    per_w = n // _NW
    small_per_w = bsz // _NW
    n_ch = per_w // ch
    n16 = d // _LANES

    mesh = plsc.VectorSubcoreMesh(core_axis_name="c", subcore_axis_name="s")

    @functools.partial(
        pl.kernel,
        out_type=(
            jax.ShapeDtypeStruct((bsz, d), jnp.float32),
            jax.ShapeDtypeStruct((_NW, d), jnp.float32),
        ),
        mesh=mesh,
        scratch_types=(
            pltpu.VMEM((small_per_w,), jnp.int32),
            pltpu.VMEM((small_per_w, d), jnp.float32),
            pltpu.VMEM((ch,), jnp.int32),
            pltpu.VMEM((ch, d), jnp.float32),
            pltpu.VMEM((d,), jnp.float32),
            pltpu.SemaphoreType.DMA,
        ),
    )
    def sc(idx_hbm, tab_hbm, emb_hbm, part_hbm,
           sidx_v, srows_v, bidx_v, brows_v, acc_v, sem):
        wid = lax.axis_index("s") * _NC + lax.axis_index("c")

        sbase = wid * small_per_w
        pltpu.sync_copy(idx_hbm.at[pl.ds(sbase, small_per_w)], sidx_v)
        pltpu.async_copy(tab_hbm.at[sidx_v], srows_v, sem).wait()
        pltpu.sync_copy(srows_v, emb_hbm.at[pl.ds(sbase, small_per_w)])

        base = wid * per_w

        def chunk(g, acc):
            off = base + g * ch
            pltpu.sync_copy(idx_hbm.at[pl.ds(off, ch)], bidx_v)
            pltpu.async_copy(tab_hbm.at[bidx_v], brows_v, sem).wait()

            def row(r, a):
                return tuple(
                    a[k] + brows_v[r, pl.ds(k * _LANES, _LANES)]
                    for k in range(n16)
                )

            return lax.fori_loop(0, ch, row, acc, unroll=8)

        zero = jnp.zeros((_LANES,), jnp.float32)
        acc = lax.fori_loop(0, n_ch, chunk, (zero,) * n16)
        for k in range(n16):
            acc_v[pl.ds(k * _LANES, _LANES)] = acc[k]
        pltpu.sync_copy(acc_v, part_hbm.at[wid])

    return sc


@functools.lru_cache(maxsize=None)
def _mlp(n, bsz, d, hidden, ncls):
    inv_cnt = 1.0 / float(n - (bsz - 1))

    def body(emb_ref, part_ref, awt_ref, ab_ref, bwt_ref, bb_ref, out_ref):
        e = emb_ref[...]
        rid = lax.broadcasted_iota(jnp.int32, e.shape, 0)
        e_masked = jnp.where(rid == bsz - 1, 0.0, e)
        total = jnp.sum(part_ref[...], axis=0)
        big = (total - jnp.sum(e_masked, axis=0)) * inv_cnt
        e2 = jnp.where(rid == bsz - 1, big[None, :], e_masked)
        h = jnp.dot(e2, awt_ref[...], preferred_element_type=jnp.float32)
        h = jnp.maximum(h + ab_ref[...], 0.0)
        out = jnp.dot(h, bwt_ref[...], preferred_element_type=jnp.float32)
        out_ref[...] = out + bb_ref[...]

    return pl.pallas_call(
        body,
        out_shape=jax.ShapeDtypeStruct((bsz, ncls), jnp.float32),
    )


def kernel(indices, offsets, table, A_w, A_b, B_w, B_b):
    n = indices.shape[0]
    bsz = offsets.shape[0]
    vocab, d = table.shape
    hidden = A_w.shape[0]
    ncls = B_w.shape[0]

    emb, part = _sc_embed(n, bsz, vocab, d, _CH)(
        indices.astype(jnp.int32), table)
    return _mlp(n, bsz, d, hidden, ncls)(
        emb, part, A_w.T, A_b[None, :], B_w.T, B_b[None, :])


# speedup vs baseline: 3.7646x; 3.7646x over previous
import functools

import jax
import jax.numpy as jnp
from jax import lax
from jax.experimental import pallas as pl
from jax.experimental.pallas import tpu as pltpu
from jax.experimental.pallas import tpu_sc as plsc

_NC = 2
_NS = 16
_NW = _NC * _NS
_LANES = 16
_CH = 512


@functools.lru_cache(maxsize=None)
def _sc_embed(n, bsz, vocab, d, ch):
    per_w = n // _NW
    small_per_w = bsz // _NW
    n_ch = per_w // ch
    n16 = d // _LANES

    mesh = plsc.VectorSubcoreMesh(core_axis_name="c", subcore_axis_name="s")

    @functools.partial(
        pl.kernel,
        out_type=(
            jax.ShapeDtypeStruct((bsz, d), jnp.float32),
            jax.ShapeDtypeStruct((_NW, d), jnp.float32),
        ),
        mesh=mesh,
        scratch_types=(
            pltpu.VMEM((small_per_w,), jnp.int32),
            pltpu.VMEM((small_per_w, d), jnp.float32),
            pltpu.VMEM((ch,), jnp.int32),
            pltpu.VMEM((ch, d), jnp.float32),
            pltpu.VMEM((d,), jnp.float32),
            pltpu.SemaphoreType.DMA,
        ),
        compiler_params=pltpu.CompilerParams(use_tc_tiling_on_sc=False),
    )
    def sc(idx_hbm, tab_hbm, emb_hbm, part_hbm,
           sidx_v, srows_v, bidx_v, brows_v, acc_v, sem):
        wid = lax.axis_index("s") * _NC + lax.axis_index("c")

        sbase = wid * small_per_w
        pltpu.sync_copy(idx_hbm.at[pl.ds(sbase, small_per_w)], sidx_v)
        pltpu.async_copy(tab_hbm.at[sidx_v], srows_v, sem).wait()
        pltpu.sync_copy(srows_v, emb_hbm.at[pl.ds(sbase, small_per_w)])

        base = wid * per_w

        def chunk(g, acc):
            off = base + g * ch
            pltpu.sync_copy(idx_hbm.at[pl.ds(off, ch)], bidx_v)
            pltpu.async_copy(tab_hbm.at[bidx_v], brows_v, sem).wait()

            def row(r, a):
                return tuple(
                    a[k] + brows_v[r, pl.ds(k * _LANES, _LANES)]
                    for k in range(n16)
                )

            return lax.fori_loop(0, ch, row, acc, unroll=8)

        zero = jnp.zeros((_LANES,), jnp.float32)
        acc = lax.fori_loop(0, n_ch, chunk, (zero,) * n16)
        for k in range(n16):
            acc_v[pl.ds(k * _LANES, _LANES)] = acc[k]
        pltpu.sync_copy(acc_v, part_hbm.at[wid])

    return sc


@functools.lru_cache(maxsize=None)
def _mlp(n, bsz, d, hidden, ncls):
    inv_cnt = 1.0 / float(n - (bsz - 1))

    def body(emb_ref, part_ref, awt_ref, ab_ref, bwt_ref, bb_ref, out_ref):
        e = emb_ref[...]
        rid = lax.broadcasted_iota(jnp.int32, e.shape, 0)
        e_masked = jnp.where(rid == bsz - 1, 0.0, e)
        total = jnp.sum(part_ref[...], axis=0)
        big = (total - jnp.sum(e_masked, axis=0)) * inv_cnt
        e2 = jnp.where(rid == bsz - 1, big[None, :], e_masked)
        h = jnp.dot(e2, awt_ref[...], preferred_element_type=jnp.float32)
        h = jnp.maximum(h + ab_ref[...], 0.0)
        out = jnp.dot(h, bwt_ref[...], preferred_element_type=jnp.float32)
        out_ref[...] = out + bb_ref[...]

    return pl.pallas_call(
        body,
        out_shape=jax.ShapeDtypeStruct((bsz, ncls), jnp.float32),
    )


def kernel(indices, offsets, table, A_w, A_b, B_w, B_b):
    n = indices.shape[0]
    bsz = offsets.shape[0]
    vocab, d = table.shape
    hidden = A_w.shape[0]
    ncls = B_w.shape[0]

    emb, part = _sc_embed(n, bsz, vocab, d, _CH)(
        indices.astype(jnp.int32), table)
    return _mlp(n, bsz, d, hidden, ncls)(
        emb, part, A_w.T, A_b[None, :], B_w.T, B_b[None, :])


# speedup vs baseline: 5.3452x; 1.4199x over previous
import functools

import jax
import jax.numpy as jnp
from jax import lax
from jax.experimental import pallas as pl
from jax.experimental.pallas import tpu as pltpu
from jax.experimental.pallas import tpu_sc as plsc

_NC = 2
_NS = 16
_NW = _NC * _NS
_LANES = 16
_CH = 512
_BLK = 1024
_S = 512000


@functools.lru_cache(maxsize=None)
def _linearize(vocab, d):
    assert d == 64 and vocab <= 2 * _S

    def body(in1_ref, in2_ref, out_ref):
        out_ref[...] = jnp.concatenate(
            [in1_ref[...].T, in2_ref[...].T], axis=1)

    last = (vocab - 1) // _BLK

    return pl.pallas_call(
        body,
        grid=(_S // _BLK,),
        in_specs=[
            pl.BlockSpec((d, _BLK), lambda p: (0, p)),
            pl.BlockSpec(
                (d, _BLK),
                lambda p: (0, jnp.minimum(_S // _BLK + p, last))),
        ],
        out_specs=pl.BlockSpec((_BLK, 2 * d), lambda p: (p, 0)),
        out_shape=jax.ShapeDtypeStruct((_S, 2 * d), jnp.float32),
    )


def _remap(v):
    return jnp.where(v < _S, 2 * v, 2 * v - (2 * _S - 1))


@functools.lru_cache(maxsize=None)
def _sc_embed(n, bsz, d):
    per_w = n // _NW
    small_per_w = bsz // _NW
    n_ch = per_w // _CH
    assert n_ch % 2 == 0
    n16 = d // _LANES

    mesh = plsc.VectorSubcoreMesh(core_axis_name="c", subcore_axis_name="s")

    @functools.partial(
        pl.kernel,
        out_type=(
            jax.ShapeDtypeStruct((bsz, d), jnp.float32),
            jax.ShapeDtypeStruct((_NW, d), jnp.float32),
        ),
        mesh=mesh,
        scratch_types=(
            pltpu.VMEM((per_w,), jnp.int32),
            pltpu.VMEM((small_per_w,), jnp.int32),
            pltpu.VMEM((small_per_w, d), jnp.float32),
            pltpu.VMEM((_CH,), jnp.int32),
            pltpu.VMEM((_CH,), jnp.int32),
            pltpu.VMEM((_CH, d), jnp.float32),
            pltpu.VMEM((_CH, d), jnp.float32),
            pltpu.VMEM((d,), jnp.float32),
            pltpu.SemaphoreType.DMA,
            pltpu.SemaphoreType.DMA,
            pltpu.SemaphoreType.DMA,
        ),
        compiler_params=pltpu.CompilerParams(use_tc_tiling_on_sc=False),
    )
    def sc(idx_hbm, flat_hbm, emb_hbm, part_hbm,
           idx_all, sidx_v, srows_v, t0, t1, b0, b1, acc_v,
           sem0, sem1, sems):
        wid = lax.axis_index("s") * _NC + lax.axis_index("c")

        sbase = wid * small_per_w
        pltpu.sync_copy(idx_hbm.at[pl.ds(sbase, small_per_w)], sidx_v)
        for j in range(small_per_w // _LANES):
            sl = pl.ds(j * _LANES, _LANES)
            sidx_v[sl] = _remap(sidx_v[sl])
        pltpu.async_copy(flat_hbm.at[sidx_v], srows_v, sems).wait()
        pltpu.sync_copy(srows_v, emb_hbm.at[pl.ds(sbase, small_per_w)])

        pltpu.sync_copy(idx_hbm.at[pl.ds(wid * per_w, per_w)], idx_all)

        def fill(g, t_ref):
            base = g * _CH
            for j in range(_CH // _LANES):
                sl = pl.ds(j * _LANES, _LANES)
                t_ref[sl] = _remap(idx_all[pl.ds(base + j * _LANES, _LANES)])

        def start(t_ref, b_ref, sem):
            pltpu.make_async_copy(flat_hbm.at[t_ref], b_ref, sem).start()

        def accum(b_ref, acc):
            def row(r, a):
                return tuple(
                    a[k] + b_ref[r, pl.ds(k * _LANES, _LANES)]
                    for k in range(n16)
                )
            return lax.fori_loop(0, _CH, row, acc, unroll=8)

        fill(0, t0)
        start(t0, b0, sem0)

        def step(i, acc):
            g = 2 * i
            fill(g + 1, t1)
            start(t1, b1, sem1)
            pltpu.make_async_copy(flat_hbm.at[t0], b0, sem0).wait()
            acc = accum(b0, acc)

            @pl.when(g + 2 < n_ch)
            def _():
                fill(g + 2, t0)
                start(t0, b0, sem0)

            pltpu.make_async_copy(flat_hbm.at[t1], b1, sem1).wait()
            return accum(b1, acc)

        zero = jnp.zeros((_LANES,), jnp.float32)
        acc = lax.fori_loop(0, n_ch // 2, step, (zero,) * n16)
        for k in range(n16):
            acc_v[pl.ds(k * _LANES, _LANES)] = acc[k]
        pltpu.sync_copy(acc_v, part_hbm.at[wid])

    return sc


@functools.lru_cache(maxsize=None)
def _mlp(n, bsz, d, hidden, ncls):
    inv_cnt = 1.0 / float(n - (bsz - 1))

    def body(emb_ref, part_ref, awt_ref, ab_ref, bwt_ref, bb_ref, out_ref):
        e = emb_ref[...]
        rid = lax.broadcasted_iota(jnp.int32, e.shape, 0)
        e_masked = jnp.where(rid == bsz - 1, 0.0, e)
        total = jnp.sum(part_ref[...], axis=0)
        big = (total - jnp.sum(e_masked, axis=0)) * inv_cnt
        e2 = jnp.where(rid == bsz - 1, big[None, :], e_masked)
        h = jnp.dot(e2, awt_ref[...], preferred_element_type=jnp.float32)
        h = jnp.maximum(h + ab_ref[...], 0.0)
        out = jnp.dot(h, bwt_ref[...], preferred_element_type=jnp.float32)
        out_ref[...] = out + bb_ref[...]

    return pl.pallas_call(
        body,
        out_shape=jax.ShapeDtypeStruct((bsz, ncls), jnp.float32),
    )


def kernel(indices, offsets, table, A_w, A_b, B_w, B_b):
    n = indices.shape[0]
    bsz = offsets.shape[0]
    vocab, d = table.shape
    hidden = A_w.shape[0]
    ncls = B_w.shape[0]

    flat2 = _linearize(vocab, d)(table.T, table.T)
    flat = flat2.reshape(2 * _S, d)
    emb, part = _sc_embed(n, bsz, d)(indices.astype(jnp.int32), flat)
    return _mlp(n, bsz, d, hidden, ncls)(
        emb, part, A_w.T, A_b[None, :], B_w.T, B_b[None, :])


# speedup vs baseline: 6.8165x; 1.2753x over previous
import functools

import jax
import jax.numpy as jnp
from jax import lax
from jax.experimental import pallas as pl
from jax.experimental.pallas import tpu as pltpu
from jax.experimental.pallas import tpu_sc as plsc

_NC = 2
_NS = 16
_NW = _NC * _NS
_LANES = 16
_CH = 512
_BLK = 2048
_S = 512000


@functools.lru_cache(maxsize=None)
def _linearize(vocab, d):
    assert d == 64 and vocab <= 2 * _S

    def body(in1_ref, in2_ref, out_ref):
        rid = lax.broadcasted_iota(jnp.int32, (d, d), 0)
        cid = lax.broadcasted_iota(jnp.int32, (d, d), 1)
        ident = jnp.where(rid == cid, 1.0, 0.0).astype(jnp.float32)
        dn = (((0,), (0,)), ((), ()))

        def tr(x):
            return lax.dot_general(
                x, ident, dn, preferred_element_type=jnp.float32)

        out_ref[...] = jnp.concatenate(
            [tr(in1_ref[...]), tr(in2_ref[...])], axis=1)

    last = (vocab - 1) // _BLK

    return pl.pallas_call(
        body,
        grid=(_S // _BLK,),
        in_specs=[
            pl.BlockSpec((d, _BLK), lambda p: (0, p)),
            pl.BlockSpec(
                (d, _BLK),
                lambda p: (0, jnp.minimum(_S // _BLK + p, last))),
        ],
        out_specs=pl.BlockSpec((_BLK, 2 * d), lambda p: (p, 0)),
        out_shape=jax.ShapeDtypeStruct((_S, 2 * d), jnp.float32),
    )


def _remap(v):
    return jnp.where(v < _S, 2 * v, 2 * v - (2 * _S - 1))


@functools.lru_cache(maxsize=None)
def _sc_embed(n, bsz, d):
    per_w = n // _NW
    small_per_w = bsz // _NW
    n_ch = per_w // _CH
    assert n_ch % 2 == 0
    n16 = d // _LANES

    mesh = plsc.VectorSubcoreMesh(core_axis_name="c", subcore_axis_name="s")

    @functools.partial(
        pl.kernel,
        out_type=(
            jax.ShapeDtypeStruct((bsz, d), jnp.float32),
            jax.ShapeDtypeStruct((_NW, d), jnp.float32),
        ),
        mesh=mesh,
        scratch_types=(
            pltpu.VMEM((per_w,), jnp.int32),
            pltpu.VMEM((small_per_w,), jnp.int32),
            pltpu.VMEM((small_per_w, d), jnp.float32),
            pltpu.VMEM((_CH,), jnp.int32),
            pltpu.VMEM((_CH,), jnp.int32),
            pltpu.VMEM((_CH, d), jnp.float32),
            pltpu.VMEM((_CH, d), jnp.float32),
            pltpu.VMEM((d,), jnp.float32),
            pltpu.SemaphoreType.DMA,
            pltpu.SemaphoreType.DMA,
            pltpu.SemaphoreType.DMA,
        ),
        compiler_params=pltpu.CompilerParams(use_tc_tiling_on_sc=False),
    )
    def sc(idx_hbm, flat_hbm, emb_hbm, part_hbm,
           idx_all, sidx_v, srows_v, t0, t1, b0, b1, acc_v,
           sem0, sem1, sems):
        wid = lax.axis_index("s") * _NC + lax.axis_index("c")

        sbase = wid * small_per_w
        pltpu.sync_copy(idx_hbm.at[pl.ds(sbase, small_per_w)], sidx_v)
        for j in range(small_per_w // _LANES):
            sl = pl.ds(j * _LANES, _LANES)
            sidx_v[sl] = _remap(sidx_v[sl])
        pltpu.async_copy(flat_hbm.at[sidx_v], srows_v, sems).wait()
        pltpu.sync_copy(srows_v, emb_hbm.at[pl.ds(sbase, small_per_w)])

        pltpu.sync_copy(idx_hbm.at[pl.ds(wid * per_w, per_w)], idx_all)

        def fill(g, t_ref):
            base = g * _CH
            for j in range(_CH // _LANES):
                sl = pl.ds(j * _LANES, _LANES)
                t_ref[sl] = _remap(idx_all[pl.ds(base + j * _LANES, _LANES)])

        def start(t_ref, b_ref, sem):
            pltpu.make_async_copy(flat_hbm.at[t_ref], b_ref, sem).start()

        def accum(b_ref, acc):
            def row(r, a):
                return tuple(
                    a[k] + b_ref[r, pl.ds(k * _LANES, _LANES)]
                    for k in range(n16)
                )
            return lax.fori_loop(0, _CH, row, acc, unroll=8)

        fill(0, t0)
        start(t0, b0, sem0)

        def step(i, acc):
            g = 2 * i
            fill(g + 1, t1)
            start(t1, b1, sem1)
            pltpu.make_async_copy(flat_hbm.at[t0], b0, sem0).wait()
            acc = accum(b0, acc)

            @pl.when(g + 2 < n_ch)
            def _():
                fill(g + 2, t0)
                start(t0, b0, sem0)

            pltpu.make_async_copy(flat_hbm.at[t1], b1, sem1).wait()
            return accum(b1, acc)

        zero = jnp.zeros((_LANES,), jnp.float32)
        acc = lax.fori_loop(0, n_ch // 2, step, (zero,) * n16)
        for k in range(n16):
            acc_v[pl.ds(k * _LANES, _LANES)] = acc[k]
        pltpu.sync_copy(acc_v, part_hbm.at[wid])

    return sc


@functools.lru_cache(maxsize=None)
def _mlp(n, bsz, d, hidden, ncls):
    inv_cnt = 1.0 / float(n - (bsz - 1))

    def body(emb_ref, part_ref, awt_ref, ab_ref, bwt_ref, bb_ref, out_ref):
        e = emb_ref[...]
        rid = lax.broadcasted_iota(jnp.int32, e.shape, 0)
        e_masked = jnp.where(rid == bsz - 1, 0.0, e)
        total = jnp.sum(part_ref[...], axis=0)
        big = (total - jnp.sum(e_masked, axis=0)) * inv_cnt
        e2 = jnp.where(rid == bsz - 1, big[None, :], e_masked)
        h = jnp.dot(e2, awt_ref[...], preferred_element_type=jnp.float32)
        h = jnp.maximum(h + ab_ref[...], 0.0)
        out = jnp.dot(h, bwt_ref[...], preferred_element_type=jnp.float32)
        out_ref[...] = out + bb_ref[...]

    return pl.pallas_call(
        body,
        out_shape=jax.ShapeDtypeStruct((bsz, ncls), jnp.float32),
    )


def kernel(indices, offsets, table, A_w, A_b, B_w, B_b):
    n = indices.shape[0]
    bsz = offsets.shape[0]
    vocab, d = table.shape
    hidden = A_w.shape[0]
    ncls = B_w.shape[0]

    flat2 = _linearize(vocab, d)(table.T, table.T)
    flat = flat2.reshape(2 * _S, d)
    emb, part = _sc_embed(n, bsz, d)(indices.astype(jnp.int32), flat)
    return _mlp(n, bsz, d, hidden, ncls)(
        emb, part, A_w.T, A_b[None, :], B_w.T, B_b[None, :])
